# baseline (speedup 1.0000x reference)
import functools

import jax
import jax.numpy as jnp
from jax import lax
from jax.experimental import pallas as pl
from jax.experimental.pallas import tpu as pltpu
from jax.experimental.pallas import tpu_sc as plsc

N = 10000
E = 320000
ND = 10240
K = 80
BLK = 1000
NSUB = 16
NP = 10240
ROWS_T = NP // NSUB
ZR = 32

_mesh = lambda: plsc.VectorSubcoreMesh(
    core_axis_name="c", subcore_axis_name="s", num_cores=2, num_subcores=NSUB)


DR = 8


def _deg_body(dst_hbm, ew_hbm, out_hbm, dacc, didx, ewv, zbuf, isem):
    c = lax.axis_index("c")
    s = lax.axis_index("s")
    w = s * 2 + c

    def zb(t, _):
        zbuf[pl.ds(t * 16, 16)] = jnp.zeros((16,), jnp.float32)
        return 0
    lax.fori_loop(0, 640 // 16, zb, 0)
    pltpu.sync_copy(zbuf, dacc.at[pl.ds(s * 640, 640)])
    plsc.subcore_barrier()

    nch = E // 64 // DR
    ntile = (nch - w + 2 * NSUB - 1) // (2 * NSUB)

    def step(i, _):
        base = (w + i * 2 * NSUB) * DR
        c1 = pltpu.async_copy(dst_hbm.at[pl.ds(base, DR)], didx, isem)
        c2 = pltpu.async_copy(ew_hbm.at[pl.ds(base, DR)], ewv, isem)
        c1.wait()
        c2.wait()
        for m in range(DR):
            pltpu.sync_copy(ewv.at[m], dacc.at[didx.at[m]], add=True)
        return 0
    lax.fori_loop(0, ntile, step, 0)
    plsc.subcore_barrier()
    pltpu.sync_copy(dacc.at[pl.ds(s * 640, 640)],
                    out_hbm.at[pl.ds(c * ND + s * 640, 640)])


def _deg(dst, ew):
    return pl.kernel(
        _deg_body,
        out_type=jax.ShapeDtypeStruct((2 * ND,), jnp.float32),
        mesh=_mesh(),
        scratch_types=[
            pltpu.VMEM_SHARED((ND,), jnp.float32),
            pltpu.VMEM((DR, 64), jnp.int32),
            pltpu.VMEM((DR, 64), jnp.float32),
            pltpu.VMEM((640,), jnp.float32),
            pltpu.SemaphoreType.DMA,
        ],
    )(dst.reshape(E // 64, 64), ew.reshape(E // 64, 64))


FH = 128


NSLOT = 4
NSLOTI = 8


def _agg_body(feat_split, g_hbm, src_hbm, dst_hbm, ew_hbm, out_hbm,
              acc, rows, sidx, didx, ewv, zbuf, *sems):
    c = lax.axis_index("c")
    s = lax.axis_index("s")
    gs = sems[:NSLOT]
    ss = sems[NSLOT:2 * NSLOT]
    isems = sems[2 * NSLOT:]

    def zb(r, _):
        for t in range(FH // 16):
            zbuf[r, pl.ds(t * 16, 16)] = jnp.zeros((16,), jnp.float32)
        return 0
    lax.fori_loop(0, ZR, zb, 0)
    zcp = []
    for q in range(ROWS_T // ZR):
        zcp.append(pltpu.async_copy(
            zbuf, acc.at[pl.ds(s * ROWS_T + q * ZR, ZR)], sems[0]))
    for cp in zcp:
        cp.wait()
    plsc.subcore_barrier()

    nch = (E if feat_split else E // 2) // K
    nt = (nch - s + NSUB - 1) // NSUB
    nt_max = (nch + NSUB - 1) // NSUB
    ebase = 0 if feat_split else c * (E // 2)

    def idx_copies(i, q):
        base = ebase + (s + i * NSUB) * K
        return (
            (src_hbm.at[pl.ds(base, K)], sidx.at[q]),
            (dst_hbm.at[pl.ds(base, K)], didx.at[q]),
            (ew_hbm.at[pl.ds(base, K)], ewv.at[q]),
        )

    def fetch_idx(i, q):
        for sr, dr in idx_copies(i, q):
            pltpu.async_copy(sr, dr, isems[q])

    def wait_idx(i, q):
        for sr, dr in idx_copies(i, q):
            pltpu.make_async_copy(sr, dr, isems[q]).wait()

    def start_gather(i, q, sl):
        wait_idx(i, q)
        if feat_split:
            def off(t, _):
                sidx[q, pl.ds(t * 16, 16)] = sidx[q, pl.ds(t * 16, 16)] * 2 + c
                return 0
            lax.fori_loop(0, K // 16, off, 0)
        pltpu.async_copy(g_hbm.at[sidx.at[q]], rows.at[sl], gs[sl])

    def wait_gather(q, sl):
        pltpu.make_async_copy(g_hbm.at[sidx.at[q]], rows.at[sl], gs[sl]).wait()

    def scatter(q, sl):
        pltpu.async_copy(rows.at[sl], acc.at[didx.at[q]], ss[sl], add=True)

    def wait_scatter(q, sl):
        pltpu.make_async_copy(rows.at[sl], acc.at[didx.at[q]], ss[sl]).wait()

    def scale(q, sl):
        def body(g, _):
            ev = ewv[q, pl.ds(g * 16, 16)]
            for l in range(16):
                j = g * 16 + l
                e = ev[l]
                for t in range(FH // 16):
                    rows[sl, j, pl.ds(t * 16, 16)] = rows[sl, j, pl.ds(t * 16, 16)] * e
            return 0
        lax.fori_loop(0, K // 16, body, 0)

    fetch_idx(0, 0)
    fetch_idx(1, 1)
    fetch_idx(2, 2)
    start_gather(0, 0, 0)
    start_gather(1, 1, 1)

    def outer(jj, _):
        for u in range(NSLOTI):
            i = jj * NSLOTI + u
            sl = u % NSLOT

            @pl.when(i < nt)
            def _():
                wait_gather(u, sl)
                q2 = (u + 2) % NSLOTI
                sl2 = (u + 2) % NSLOT

                @pl.when(i + 2 < nt)
                def _():
                    @pl.when(i >= 2)
                    def _():
                        wait_scatter(q2, sl2)
                    start_gather(i + 2, q2, sl2)

                @pl.when(i + 3 < nt)
                def _():
                    fetch_idx(i + 3, (u + 3) % NSLOTI)

                scale(u, sl)
                scatter(u, sl)
        return 0
    lax.fori_loop(0, (nt_max + NSLOTI - 1) // NSLOTI, outer, 0)
    for u in range(NSLOT):
        wait_scatter(0, u)
    plsc.subcore_barrier()
    pltpu.sync_copy(acc.at[pl.ds(s * ROWS_T, ROWS_T)],
                    out_hbm.at[c, pl.ds(s * ROWS_T, ROWS_T)])


def _agg(g, src, dst, ew, feat_split):
    return pl.kernel(
        functools.partial(_agg_body, feat_split),
        out_type=jax.ShapeDtypeStruct((2, NP, FH), jnp.float32),
        mesh=_mesh(),
        scratch_types=[
            pltpu.VMEM_SHARED((NP, FH), jnp.float32),
            pltpu.VMEM((NSLOT, K, FH), jnp.float32),
            pltpu.VMEM((NSLOTI, K), jnp.int32),
            pltpu.VMEM((NSLOTI, K), jnp.int32),
            pltpu.VMEM((NSLOTI, K), jnp.float32),
            pltpu.VMEM((ZR, FH), jnp.float32),
        ] + [pltpu.SemaphoreType.DMA] * (2 * NSLOT + NSLOTI),
    )(g, src, dst, ew)


def _dinv_g1_kernel(deg_ref, x_ref, dinv_ref, g_ref):
    d = deg_ref[0] + deg_ref[1]
    safe = jnp.where(d > 0, d, 1.0)
    dv = jnp.where(d > 0, lax.rsqrt(safe), 0.0)
    dinv_ref[:] = dv
    g_ref[:] = x_ref[:] * dv


def _dinv_g1(deg2, x):
    f = x.shape[1]
    return pl.pallas_call(
        _dinv_g1_kernel,
        grid=(N // BLK,),
        in_specs=[
            pl.BlockSpec((2, BLK, 1), lambda i: (0, i, 0)),
            pl.BlockSpec((BLK, f), lambda i: (i, 0)),
        ],
        out_specs=(pl.BlockSpec((BLK, 1), lambda i: (i, 0)),
                   pl.BlockSpec((BLK, f), lambda i: (i, 0))),
        out_shape=(jax.ShapeDtypeStruct((ND, 1), jnp.float32),
                   jax.ShapeDtypeStruct((N, f), jnp.float32)),
    )(deg2.reshape(2, ND, 1), x)


def _root_kernel(x_ref, w_ref, b_ref, out_ref):
    out_ref[:] = (jnp.dot(x_ref[:], w_ref[:], preferred_element_type=jnp.float32)
                  + b_ref[:])


def _root(x, w, b):
    fi = x.shape[1]
    fo = w.shape[1]
    return pl.pallas_call(
        _root_kernel,
        grid=(N // BLK,),
        in_specs=[
            pl.BlockSpec((BLK, fi), lambda i: (i, 0)),
            pl.BlockSpec((fi, fo), lambda i: (0, 0)),
            pl.BlockSpec((1, fo), lambda i: (0, 0)),
        ],
        out_specs=pl.BlockSpec((BLK, fo), lambda i: (i, 0)),
        out_shape=jax.ShapeDtypeStruct((N, fo), jnp.float32),
    )(x, w, b)


def _combine_kernel(concat, pre_mm, has_next, *refs):
    if pre_mm:
        a0_ref, a1_ref, dinv_ref, wi_ref, r_ref = refs[:5]
    else:
        a0_ref, a1_ref, dinv_ref, r_ref = refs[:4]
    if concat:
        agg = jnp.concatenate([a0_ref[0], a1_ref[0]], axis=1)
    else:
        agg = a0_ref[0] + a1_ref[0]
    agg = agg * dinv_ref[:]
    if pre_mm:
        agg = jnp.dot(agg, wi_ref[:], preferred_element_type=jnp.float32)
    h = jnp.maximum(agg + r_ref[:], 0.0)
    if has_next:
        win_ref, h_ref, g_ref = refs[-3:]
        h_ref[:] = h
        g_ref[:] = jnp.dot(h, win_ref[:], preferred_element_type=jnp.float32) * dinv_ref[:]
    else:
        refs[-1][:] = h


def _combine(aggs, dinv, r, concat, wi_pre=None, wi_next=None):
    fo = r.shape[1]
    in_specs = [
        pl.BlockSpec((1, BLK, FH), lambda i: (0, i, 0)),
        pl.BlockSpec((1, BLK, FH), lambda i: (1, i, 0)),
        pl.BlockSpec((BLK, 1), lambda i: (i, 0)),
    ]
    args = [aggs, aggs, dinv]
    if wi_pre is not None:
        in_specs.append(pl.BlockSpec((FH, fo), lambda i: (0, 0)))
        args.append(wi_pre)
    in_specs.append(pl.BlockSpec((BLK, fo), lambda i: (i, 0)))
    args.append(r)
    out_shape = jax.ShapeDtypeStruct((N, fo), jnp.float32)
    out_spec = pl.BlockSpec((BLK, fo), lambda i: (i, 0))
    if wi_next is not None:
        fn = wi_next.shape[1]
        in_specs.append(pl.BlockSpec((fo, fn), lambda i: (0, 0)))
        args.append(wi_next)
        out_shape = (out_shape, jax.ShapeDtypeStruct((N, fn), jnp.float32))
        out_spec = (out_spec, pl.BlockSpec((BLK, fn), lambda i: (i, 0)))
    return pl.pallas_call(
        functools.partial(_combine_kernel, concat, wi_pre is not None,
                          wi_next is not None),
        grid=(N // BLK,),
        in_specs=in_specs,
        out_specs=out_spec,
        out_shape=out_shape,
    )(*args)


def kernel(x, edge_index, edge_attr, W_init1, W_root1, b1, W_init2, W_root2, b2,
           W_init3, W_root3, b3, W_init4, W_root4, b4):
    src = edge_index[0]
    dst = edge_index[1]
    ew = edge_attr

    deg2 = _deg(dst, ew)

    dinv, g1 = _dinv_g1(deg2, x)
    r1 = _root(x, W_root1, b1.reshape(1, -1))
    s1 = _agg(g1, src, dst, ew, feat_split=False)
    h1, g2 = _combine(s1, dinv, r1, concat=False, wi_pre=W_init1, wi_next=W_init2)

    r2 = _root(h1, W_root2, b2.reshape(1, -1))
    s2 = _agg(g2.reshape(2 * N, FH), src, dst, ew, feat_split=True)
    h2, g3 = _combine(s2, dinv, r2, concat=True, wi_next=W_init3)

    r3 = _root(h2, W_root3, b3.reshape(1, -1))
    s3 = _agg(g3.reshape(2 * N, FH), src, dst, ew, feat_split=True)
    h3, g4 = _combine(s3, dinv, r3, concat=True, wi_next=W_init4)

    r4 = _root(h3, W_root4, b4.reshape(1, -1))
    s4 = _agg(g4, src, dst, ew, feat_split=False)
    h4 = _combine(s4, dinv, r4, concat=False)
    return h4


# speedup vs baseline: 1.0005x; 1.0005x over previous
import functools

import jax
import jax.numpy as jnp
from jax import lax
from jax.experimental import pallas as pl
from jax.experimental.pallas import tpu as pltpu
from jax.experimental.pallas import tpu_sc as plsc

N = 10000
E = 320000
ND = 10240
K = 80
BLK = 1000
NSUB = 16
NP = 10240
ROWS_T = NP // NSUB
ZR = 32

_mesh = lambda: plsc.VectorSubcoreMesh(
    core_axis_name="c", subcore_axis_name="s", num_cores=2, num_subcores=NSUB)


DR = 8


def _deg_body(dst_hbm, ew_hbm, out_hbm, dacc, didx, ewv, zbuf, isem):
    c = lax.axis_index("c")
    s = lax.axis_index("s")
    w = s * 2 + c

    def zb(t, _):
        zbuf[pl.ds(t * 16, 16)] = jnp.zeros((16,), jnp.float32)
        return 0
    lax.fori_loop(0, 640 // 16, zb, 0)
    pltpu.sync_copy(zbuf, dacc.at[pl.ds(s * 640, 640)])
    plsc.subcore_barrier()

    nch = E // 64 // DR
    ntile = (nch - w + 2 * NSUB - 1) // (2 * NSUB)

    def step(i, _):
        base = (w + i * 2 * NSUB) * DR
        c1 = pltpu.async_copy(dst_hbm.at[pl.ds(base, DR)], didx, isem)
        c2 = pltpu.async_copy(ew_hbm.at[pl.ds(base, DR)], ewv, isem)
        c1.wait()
        c2.wait()
        for m in range(DR):
            pltpu.sync_copy(ewv.at[m], dacc.at[didx.at[m]], add=True)
        return 0
    lax.fori_loop(0, ntile, step, 0)
    plsc.subcore_barrier()
    pltpu.sync_copy(dacc.at[pl.ds(s * 640, 640)],
                    out_hbm.at[pl.ds(c * ND + s * 640, 640)])


def _deg(dst, ew):
    return pl.kernel(
        _deg_body,
        out_type=jax.ShapeDtypeStruct((2 * ND,), jnp.float32),
        mesh=_mesh(),
        scratch_types=[
            pltpu.VMEM_SHARED((ND,), jnp.float32),
            pltpu.VMEM((DR, 64), jnp.int32),
            pltpu.VMEM((DR, 64), jnp.float32),
            pltpu.VMEM((640,), jnp.float32),
            pltpu.SemaphoreType.DMA,
        ],
    )(dst.reshape(E // 64, 64), ew.reshape(E // 64, 64))


FH = 128


NSLOT = 4
NSLOTI = 8


def _agg_body(feat_split, g_hbm, src_hbm, dst_hbm, ew_hbm, out_hbm,
              acc, rows, sidx, didx, ewv, zbuf, *sems):
    c = lax.axis_index("c")
    s = lax.axis_index("s")
    gs = sems[:NSLOT]
    ss = sems[NSLOT:2 * NSLOT]
    isems = sems[2 * NSLOT:]

    def zb(r, _):
        for t in range(FH // 16):
            zbuf[r, pl.ds(t * 16, 16)] = jnp.zeros((16,), jnp.float32)
        return 0
    lax.fori_loop(0, ZR, zb, 0)
    zcp = []
    for q in range(ROWS_T // ZR):
        zcp.append(pltpu.async_copy(
            zbuf, acc.at[pl.ds(s * ROWS_T + q * ZR, ZR)], sems[0]))
    for cp in zcp:
        cp.wait()
    plsc.subcore_barrier()

    nch = (E if feat_split else E // 2) // K
    nt = (nch - s + NSUB - 1) // NSUB
    nt_max = (nch + NSUB - 1) // NSUB
    ebase = 0 if feat_split else c * (E // 2)

    def idx_copies(i, q):
        base = ebase + (s + i * NSUB) * K
        return (
            (src_hbm.at[pl.ds(base, K)], sidx.at[q]),
            (dst_hbm.at[pl.ds(base, K)], didx.at[q]),
            (ew_hbm.at[pl.ds(base, K)], ewv.at[q]),
        )

    def fetch_idx(i, q):
        for sr, dr in idx_copies(i, q):
            pltpu.async_copy(sr, dr, isems[q])

    def wait_idx(i, q):
        for sr, dr in idx_copies(i, q):
            pltpu.make_async_copy(sr, dr, isems[q]).wait()

    def start_gather(i, q, sl):
        wait_idx(i, q)
        if feat_split:
            def off(t, _):
                sidx[q, pl.ds(t * 16, 16)] = sidx[q, pl.ds(t * 16, 16)] * 2 + c
                return 0
            lax.fori_loop(0, K // 16, off, 0)
        pltpu.async_copy(g_hbm.at[sidx.at[q]], rows.at[sl], gs[sl])

    def wait_gather(q, sl):
        pltpu.make_async_copy(g_hbm.at[sidx.at[q]], rows.at[sl], gs[sl]).wait()

    def scatter(q, sl):
        pltpu.async_copy(rows.at[sl], acc.at[didx.at[q]], ss[sl], add=True)

    def wait_scatter(q, sl):
        pltpu.make_async_copy(rows.at[sl], acc.at[didx.at[q]], ss[sl]).wait()

    def scale(q, sl):
        def body(g, _):
            ev = ewv[q, pl.ds(g * 16, 16)]
            for l in range(16):
                j = g * 16 + l
                e = ev[l]
                for t in range(FH // 16):
                    rows[sl, j, pl.ds(t * 16, 16)] = rows[sl, j, pl.ds(t * 16, 16)] * e
            return 0
        lax.fori_loop(0, K // 16, body, 0)

    fetch_idx(0, 0)
    fetch_idx(1, 1)
    fetch_idx(2, 2)
    start_gather(0, 0, 0)
    start_gather(1, 1, 1)

    def outer(jj, _):
        for u in range(NSLOTI):
            i = jj * NSLOTI + u
            sl = u % NSLOT

            @pl.when(i < nt)
            def _():
                wait_gather(u, sl)
                q2 = (u + 2) % NSLOTI
                sl2 = (u + 2) % NSLOT

                @pl.when(i + 2 < nt)
                def _():
                    @pl.when(i >= 2)
                    def _():
                        wait_scatter(q2, sl2)
                    start_gather(i + 2, q2, sl2)

                @pl.when(i + 3 < nt)
                def _():
                    fetch_idx(i + 3, (u + 3) % NSLOTI)

                scale(u, sl)
                scatter(u, sl)
        return 0
    lax.fori_loop(0, (nt_max + NSLOTI - 1) // NSLOTI, outer, 0)
    for u in range(NSLOT):
        wait_scatter(0, u)
    plsc.subcore_barrier()
    pltpu.sync_copy(acc.at[pl.ds(s * ROWS_T, ROWS_T)],
                    out_hbm.at[c, pl.ds(s * ROWS_T, ROWS_T)])


def _agg(g, src, dst, ew, feat_split):
    return pl.kernel(
        functools.partial(_agg_body, feat_split),
        out_type=jax.ShapeDtypeStruct((2, NP, FH), jnp.float32),
        mesh=_mesh(),
        scratch_types=[
            pltpu.VMEM_SHARED((NP, FH), jnp.float32),
            pltpu.VMEM((NSLOT, K, FH), jnp.float32),
            pltpu.VMEM((NSLOTI, K), jnp.int32),
            pltpu.VMEM((NSLOTI, K), jnp.int32),
            pltpu.VMEM((NSLOTI, K), jnp.float32),
            pltpu.VMEM((ZR, FH), jnp.float32),
        ] + [pltpu.SemaphoreType.DMA] * (2 * NSLOT + NSLOTI),
    )(g, src, dst, ew)


def _dinv_g1_kernel(deg_ref, x_ref, wr_ref, b_ref, dinv_ref, g_ref, r_ref):
    d = deg_ref[0] + deg_ref[1]
    safe = jnp.where(d > 0, d, 1.0)
    dv = jnp.where(d > 0, lax.rsqrt(safe), 0.0)
    dinv_ref[:] = dv
    g_ref[:] = x_ref[:] * dv
    r_ref[:] = (jnp.dot(x_ref[:], wr_ref[:], preferred_element_type=jnp.float32)
                + b_ref[:])


def _dinv_g1(deg2, x, wr, b):
    f = x.shape[1]
    fo = wr.shape[1]
    return pl.pallas_call(
        _dinv_g1_kernel,
        grid=(N // BLK,),
        in_specs=[
            pl.BlockSpec((2, BLK, 1), lambda i: (0, i, 0)),
            pl.BlockSpec((BLK, f), lambda i: (i, 0)),
            pl.BlockSpec((f, fo), lambda i: (0, 0)),
            pl.BlockSpec((1, fo), lambda i: (0, 0)),
        ],
        out_specs=(pl.BlockSpec((BLK, 1), lambda i: (i, 0)),
                   pl.BlockSpec((BLK, f), lambda i: (i, 0)),
                   pl.BlockSpec((BLK, fo), lambda i: (i, 0))),
        out_shape=(jax.ShapeDtypeStruct((ND, 1), jnp.float32),
                   jax.ShapeDtypeStruct((N, f), jnp.float32),
                   jax.ShapeDtypeStruct((N, fo), jnp.float32)),
    )(deg2.reshape(2, ND, 1), x, wr, b)


def _combine_kernel(concat, pre_mm, has_next, *refs):
    if pre_mm:
        a0_ref, a1_ref, dinv_ref, wi_ref, r_ref = refs[:5]
        rest = refs[5:]
    else:
        a0_ref, a1_ref, dinv_ref, r_ref = refs[:4]
        rest = refs[4:]
    if concat:
        agg = jnp.concatenate([a0_ref[0], a1_ref[0]], axis=1)
    else:
        agg = a0_ref[0] + a1_ref[0]
    agg = agg * dinv_ref[:]
    if pre_mm:
        agg = jnp.dot(agg, wi_ref[:], preferred_element_type=jnp.float32)
    h = jnp.maximum(agg + r_ref[:], 0.0)
    if has_next:
        win_ref, wrn_ref, bn_ref, g_ref, rn_ref = rest
        g_ref[:] = jnp.dot(h, win_ref[:], preferred_element_type=jnp.float32) * dinv_ref[:]
        rn_ref[:] = (jnp.dot(h, wrn_ref[:], preferred_element_type=jnp.float32)
                     + bn_ref[:])
    else:
        rest[-1][:] = h


def _combine(aggs, dinv, r, concat, wi_pre=None, nxt=None):
    fo = r.shape[1]
    in_specs = [
        pl.BlockSpec((1, BLK, FH), lambda i: (0, i, 0)),
        pl.BlockSpec((1, BLK, FH), lambda i: (1, i, 0)),
        pl.BlockSpec((BLK, 1), lambda i: (i, 0)),
    ]
    args = [aggs, aggs, dinv]
    if wi_pre is not None:
        in_specs.append(pl.BlockSpec((FH, fo), lambda i: (0, 0)))
        args.append(wi_pre)
    in_specs.append(pl.BlockSpec((BLK, fo), lambda i: (i, 0)))
    args.append(r)
    if nxt is not None:
        wi_next, wr_next, b_next = nxt
        fn = wi_next.shape[1]
        fr = wr_next.shape[1]
        in_specs += [
            pl.BlockSpec((fo, fn), lambda i: (0, 0)),
            pl.BlockSpec((fo, fr), lambda i: (0, 0)),
            pl.BlockSpec((1, fr), lambda i: (0, 0)),
        ]
        args += [wi_next, wr_next, b_next]
        out_shape = (jax.ShapeDtypeStruct((N, fn), jnp.float32),
                     jax.ShapeDtypeStruct((N, fr), jnp.float32))
        out_spec = (pl.BlockSpec((BLK, fn), lambda i: (i, 0)),
                    pl.BlockSpec((BLK, fr), lambda i: (i, 0)))
    else:
        out_shape = jax.ShapeDtypeStruct((N, fo), jnp.float32)
        out_spec = pl.BlockSpec((BLK, fo), lambda i: (i, 0))
    return pl.pallas_call(
        functools.partial(_combine_kernel, concat, wi_pre is not None,
                          nxt is not None),
        grid=(N // BLK,),
        in_specs=in_specs,
        out_specs=out_spec,
        out_shape=out_shape,
    )(*args)


def kernel(x, edge_index, edge_attr, W_init1, W_root1, b1, W_init2, W_root2, b2,
           W_init3, W_root3, b3, W_init4, W_root4, b4):
    src = edge_index[0]
    dst = edge_index[1]
    ew = edge_attr

    deg2 = _deg(dst, ew)

    dinv, g1, r1 = _dinv_g1(deg2, x, W_root1, b1.reshape(1, -1))
    s1 = _agg(g1, src, dst, ew, feat_split=False)
    g2, r2 = _combine(s1, dinv, r1, concat=False, wi_pre=W_init1,
                      nxt=(W_init2, W_root2, b2.reshape(1, -1)))

    s2 = _agg(g2.reshape(2 * N, FH), src, dst, ew, feat_split=True)
    g3, r3 = _combine(s2, dinv, r2, concat=True,
                      nxt=(W_init3, W_root3, b3.reshape(1, -1)))

    s3 = _agg(g3.reshape(2 * N, FH), src, dst, ew, feat_split=True)
    g4, r4 = _combine(s3, dinv, r3, concat=True,
                      nxt=(W_init4, W_root4, b4.reshape(1, -1)))

    s4 = _agg(g4, src, dst, ew, feat_split=False)
    return _combine(s4, dinv, r4, concat=False)
